# Initial kernel scaffold; baseline (speedup 1.0000x reference)
import jax
import jax.numpy as jnp
from jax.experimental import pallas as pl


def kernel(x, edge_index, edge_weight, Wz, bz, Lz, lbz, Wr, br, Lr, lbr, Wh, bh, Lh, lbh, att, lin_w, lin_b):
    raise NotImplementedError("write your pallas kernel here")


# speedup vs baseline: 1.9136x; 1.9136x over previous
import functools

import jax
import jax.numpy as jnp
from jax import lax
from jax.experimental import pallas as pl
from jax.experimental.pallas import tpu as pltpu

F = 128
PT = 12
ROW_BLK = 512


def _gru_block(parts_ref, xt_ref, dinv_ref, probs_ref,
               Wz_ref, Lz_ref, bz_ref, lbz_ref,
               Wr_ref, Lr_ref, br_ref, lbr_ref,
               Wh_ref, Lh_ref, bh_ref, lbh_ref,
               lin_w_ref, lin_b_ref, out_ref):
    C = parts_ref.shape[0]
    f32 = jnp.float32

    def fold(W_ref, L_ref, b_ref, lb_ref):
        Lt = L_ref[0:F, :]
        Lb = L_ref[F:2 * F, :]
        WL = jnp.dot(W_ref[...], Lt, preferred_element_type=f32)
        c = jnp.dot(b_ref[...], Lt, preferred_element_type=f32) + lb_ref[...]
        return WL, Lb, c

    WLz, Lzb, cz = fold(Wz_ref, Lz_ref, bz_ref, lbz_ref)
    WLr, Lrb, cr = fold(Wr_ref, Lr_ref, br_ref, lbr_ref)
    WLh, Lhb, ch = fold(Wh_ref, Lh_ref, bh_ref, lbh_ref)

    dv = dinv_ref[...]
    R = dv.shape[0]
    H = jnp.zeros((R, F), dtype=f32)
    Hacc = jnp.zeros((R, F), dtype=f32)
    for t in range(PT):
        s = parts_ref[0, t]
        for c in range(1, C):
            s = s + parts_ref[c, t]
        a = dv * (s + dv * xt_ref[t])
        Z = jax.nn.sigmoid(jnp.dot(a, WLz, preferred_element_type=f32)
                           + jnp.dot(H, Lzb, preferred_element_type=f32) + cz)
        Rg = jax.nn.sigmoid(jnp.dot(a, WLr, preferred_element_type=f32)
                            + jnp.dot(H, Lrb, preferred_element_type=f32) + cr)
        Ht = jnp.tanh(jnp.dot(a, WLh, preferred_element_type=f32)
                      + jnp.dot(H * Rg, Lhb, preferred_element_type=f32) + ch)
        H = Z * H + (1.0 - Z) * Ht
        Hacc = Hacc + probs_ref[t] * H
    out_ref[...] = (jnp.dot(jnp.maximum(Hacc, 0.0), lin_w_ref[...],
                            preferred_element_type=f32) + lin_b_ref[...])


def _gru_pallas(parts, xt, dinv2d, probs,
                Wz, Lz, bz, lbz, Wr, Lr, br, lbr, Wh, Lh, bh, lbh,
                lin_w, lin_b):
    C = parts.shape[0]
    n = xt.shape[1]
    grid = (pl.cdiv(n, ROW_BLK),)
    full = lambda shape: pl.BlockSpec(shape, lambda i: (0,) * len(shape))
    return pl.pallas_call(
        _gru_block,
        grid=grid,
        in_specs=[
            pl.BlockSpec((C, PT, ROW_BLK, F), lambda i: (0, 0, i, 0)),
            pl.BlockSpec((PT, ROW_BLK, F), lambda i: (0, i, 0)),
            pl.BlockSpec((ROW_BLK, 1), lambda i: (i, 0)),
            pl.BlockSpec(memory_space=pltpu.SMEM),
            full((F, F)), full((2 * F, F)), full((1, F)), full((1, F)),
            full((F, F)), full((2 * F, F)), full((1, F)), full((1, F)),
            full((F, F)), full((2 * F, F)), full((1, F)), full((1, F)),
            full((F, PT)), full((1, PT)),
        ],
        out_specs=pl.BlockSpec((ROW_BLK, PT), lambda i: (i, 0)),
        out_shape=jax.ShapeDtypeStruct((n, PT), jnp.float32),
    )(parts, xt, dinv2d, probs,
      Wz, Lz, bz, lbz, Wr, Lr, br, lbr, Wh, Lh, bh, lbh, lin_w, lin_b)


def kernel(x, edge_index, edge_weight, Wz, bz, Lz, lbz, Wr, br, Lr, lbr,
           Wh, bh, Lh, lbh, att, lin_w, lin_b):
    n = x.shape[0]
    src = edge_index[0]
    dst = edge_index[1]

    deg = jnp.zeros((n,), jnp.float32).at[dst].add(edge_weight) + 1.0
    dinv = jnp.where(deg > 0, lax.rsqrt(jnp.where(deg > 0, deg, 1.0)), 0.0)

    xt_all = jnp.transpose(x, (2, 0, 1))
    we = edge_weight * dinv[src]

    def one_period(xt):
        return jnp.zeros((n, F), jnp.float32).at[dst].add(xt[src] * we[:, None])
    S = jax.vmap(one_period)(xt_all)
    parts = S[None]

    probs = jax.nn.softmax(att)
    out = _gru_pallas(parts, xt_all, dinv.reshape(n, 1), probs,
                      Wz, Lz, bz.reshape(1, F), lbz.reshape(1, F),
                      Wr, Lr, br.reshape(1, F), lbr.reshape(1, F),
                      Wh, Lh, bh.reshape(1, F), lbh.reshape(1, F),
                      lin_w, lin_b.reshape(1, PT))
    return out


# speedup vs baseline: 15.5773x; 8.1402x over previous
import functools

import jax
import jax.numpy as jnp
from jax import lax
from jax.experimental import pallas as pl
from jax.experimental.pallas import tpu as pltpu
from jax.experimental.pallas import tpu_sc as plsc

F = 128
PT = 12
ROW_BLK = 512

N_NODES = 10000
N_PAD = 10240
E_EDGES = 320000
NS = 16
NC = 2
EDGES_PER_TILE = E_EDGES // NS
CHUNK = 80
EBLK = 2000
TP_PER_CORE = PT // NC
ROWS_PER_TILE = N_PAD // NS
OCHUNK = 128


def _sc_agg_body(xt_hbm, src_hbm, dst_hbm, we_hbm, out_hbm,
                 srcb, dstb, web, gidx, sidx, rows, zbuf, accum, sem):
    c = lax.axis_index("c")
    s = lax.axis_index("s")
    ebase = s * EDGES_PER_TILE

    z16 = jnp.zeros((16,), jnp.float32)

    def zero_zbuf(r, _):
        for j in range(F // 16):
            zbuf[r, pl.ds(j * 16, 16)] = z16
        return 0

    lax.fori_loop(0, OCHUNK, zero_zbuf, 0)
    for k in range(ROWS_PER_TILE // OCHUNK):
        pltpu.sync_copy(zbuf, accum.at[pl.ds(s * ROWS_PER_TILE + k * OCHUNK,
                                             OCHUNK)])
    plsc.subcore_barrier()

    def period_body(tp, _):
        tg = c * TP_PER_CORE + tp
        toff = tg * N_NODES

        def block_body(b, _):
            bb = ebase + b * EBLK
            pltpu.sync_copy(src_hbm.at[pl.ds(bb, EBLK)], srcb)
            pltpu.sync_copy(dst_hbm.at[pl.ds(bb, EBLK)], dstb)
            pltpu.sync_copy(we_hbm.at[pl.ds(bb, EBLK)], web)

            def chunk_body(ci, _):
                eb = ci * CHUNK
                for j in range(CHUNK // 16):
                    sl = pl.ds(j * 16, 16)
                    esl = pl.ds(eb + j * 16, 16)
                    gidx[sl] = srcb[esl] + toff
                    sidx[sl] = dstb[esl]
                pltpu.async_copy(xt_hbm.at[gidx], rows, sem).wait()
                for g in range(CHUNK // 16):
                    w16 = web[pl.ds(eb + g * 16, 16)]
                    for l in range(16):
                        w = w16[l]
                        i = g * 16 + l
                        for j in range(F // 16):
                            sl = pl.ds(j * 16, 16)
                            rows[i, sl] = rows[i, sl] * w
                pltpu.sync_copy(rows, accum.at[sidx], add=True)
                return 0

            lax.fori_loop(0, EBLK // CHUNK, chunk_body, 0)
            return 0

        lax.fori_loop(0, EDGES_PER_TILE // EBLK, block_body, 0)
        plsc.subcore_barrier()
        for k in range(ROWS_PER_TILE // OCHUNK):
            r0 = s * ROWS_PER_TILE + k * OCHUNK
            pltpu.sync_copy(accum.at[pl.ds(r0, OCHUNK)],
                            out_hbm.at[tg, pl.ds(r0, OCHUNK)])
            pltpu.sync_copy(zbuf, accum.at[pl.ds(r0, OCHUNK)])
        plsc.subcore_barrier()
        return 0

    lax.fori_loop(0, TP_PER_CORE, period_body, 0)


def _sc_aggregate(xt_flat, src, dst, we):
    mesh = plsc.VectorSubcoreMesh(core_axis_name="c", subcore_axis_name="s")
    fn = functools.partial(
        pl.kernel,
        mesh=mesh,
        out_type=jax.ShapeDtypeStruct((PT, N_PAD, F), jnp.float32),
        scratch_types=[
            pltpu.VMEM((EBLK,), jnp.int32),
            pltpu.VMEM((EBLK,), jnp.int32),
            pltpu.VMEM((EBLK,), jnp.float32),
            pltpu.VMEM((CHUNK,), jnp.int32),
            pltpu.VMEM((CHUNK,), jnp.int32),
            pltpu.VMEM((CHUNK, F), jnp.float32),
            pltpu.VMEM((OCHUNK, F), jnp.float32),
            pltpu.VMEM_SHARED((N_PAD, F), jnp.float32),
            pltpu.SemaphoreType.DMA,
        ],
    )(_sc_agg_body)
    return fn(xt_flat, src, dst, we)


def _gru_block(parts_ref, xt_ref, dinv_ref, probs_ref,
               Wz_ref, Lz_ref, bz_ref, lbz_ref,
               Wr_ref, Lr_ref, br_ref, lbr_ref,
               Wh_ref, Lh_ref, bh_ref, lbh_ref,
               lin_w_ref, lin_b_ref, out_ref):
    C = parts_ref.shape[0]
    f32 = jnp.float32

    def fold(W_ref, L_ref, b_ref, lb_ref):
        Lt = L_ref[0:F, :]
        Lb = L_ref[F:2 * F, :]
        WL = jnp.dot(W_ref[...], Lt, preferred_element_type=f32)
        c = jnp.dot(b_ref[...], Lt, preferred_element_type=f32) + lb_ref[...]
        return WL, Lb, c

    WLz, Lzb, cz = fold(Wz_ref, Lz_ref, bz_ref, lbz_ref)
    WLr, Lrb, cr = fold(Wr_ref, Lr_ref, br_ref, lbr_ref)
    WLh, Lhb, ch = fold(Wh_ref, Lh_ref, bh_ref, lbh_ref)

    dv = dinv_ref[...]
    R = dv.shape[0]
    H = jnp.zeros((R, F), dtype=f32)
    Hacc = jnp.zeros((R, F), dtype=f32)
    for t in range(PT):
        s = parts_ref[0, t]
        for c in range(1, C):
            s = s + parts_ref[c, t]
        a = dv * (s + dv * xt_ref[t])
        Z = jax.nn.sigmoid(jnp.dot(a, WLz, preferred_element_type=f32)
                           + jnp.dot(H, Lzb, preferred_element_type=f32) + cz)
        Rg = jax.nn.sigmoid(jnp.dot(a, WLr, preferred_element_type=f32)
                            + jnp.dot(H, Lrb, preferred_element_type=f32) + cr)
        Ht = jnp.tanh(jnp.dot(a, WLh, preferred_element_type=f32)
                      + jnp.dot(H * Rg, Lhb, preferred_element_type=f32) + ch)
        H = Z * H + (1.0 - Z) * Ht
        Hacc = Hacc + probs_ref[t] * H
    out_ref[...] = (jnp.dot(jnp.maximum(Hacc, 0.0), lin_w_ref[...],
                            preferred_element_type=f32) + lin_b_ref[...])


def _gru_pallas(parts, xt, dinv2d, probs,
                Wz, Lz, bz, lbz, Wr, Lr, br, lbr, Wh, Lh, bh, lbh,
                lin_w, lin_b):
    C = parts.shape[0]
    n = xt.shape[1]
    grid = (pl.cdiv(n, ROW_BLK),)
    full = lambda shape: pl.BlockSpec(shape, lambda i: (0,) * len(shape))
    return pl.pallas_call(
        _gru_block,
        grid=grid,
        in_specs=[
            pl.BlockSpec((C, PT, ROW_BLK, F), lambda i: (0, 0, i, 0)),
            pl.BlockSpec((PT, ROW_BLK, F), lambda i: (0, i, 0)),
            pl.BlockSpec((ROW_BLK, 1), lambda i: (i, 0)),
            pl.BlockSpec(memory_space=pltpu.SMEM),
            full((F, F)), full((2 * F, F)), full((1, F)), full((1, F)),
            full((F, F)), full((2 * F, F)), full((1, F)), full((1, F)),
            full((F, F)), full((2 * F, F)), full((1, F)), full((1, F)),
            full((F, PT)), full((1, PT)),
        ],
        out_specs=pl.BlockSpec((ROW_BLK, PT), lambda i: (i, 0)),
        out_shape=jax.ShapeDtypeStruct((n, PT), jnp.float32),
    )(parts, xt, dinv2d, probs,
      Wz, Lz, bz, lbz, Wr, Lr, br, lbr, Wh, Lh, bh, lbh, lin_w, lin_b)


def kernel(x, edge_index, edge_weight, Wz, bz, Lz, lbz, Wr, br, Lr, lbr,
           Wh, bh, Lh, lbh, att, lin_w, lin_b):
    n = x.shape[0]
    src = edge_index[0]
    dst = edge_index[1]

    deg = jnp.zeros((n,), jnp.float32).at[dst].add(edge_weight) + 1.0
    dinv = jnp.where(deg > 0, lax.rsqrt(jnp.where(deg > 0, deg, 1.0)), 0.0)

    xt_all = jnp.transpose(x, (2, 0, 1))
    we = edge_weight * dinv[src]

    S = _sc_aggregate(xt_all.reshape(PT * n, F), src, dst, we)
    parts = S[None]

    probs = jax.nn.softmax(att)
    out = _gru_pallas(parts, xt_all, dinv.reshape(n, 1), probs,
                      Wz, Lz, bz.reshape(1, F), lbz.reshape(1, F),
                      Wr, Lr, br.reshape(1, F), lbr.reshape(1, F),
                      Wh, Lh, bh.reshape(1, F), lbh.reshape(1, F),
                      lin_w, lin_b.reshape(1, PT))
    return out


# speedup vs baseline: 19.8421x; 1.2738x over previous
import functools

import jax
import jax.numpy as jnp
from jax import lax
from jax.experimental import pallas as pl
from jax.experimental.pallas import tpu as pltpu
from jax.experimental.pallas import tpu_sc as plsc

F = 128
PT = 12
ROW_BLK = 512

N_NODES = 10000
N_PAD = 10240
E_EDGES = 320000
NS = 16
NC = 2
EDGES_PER_TILE = E_EDGES // NS
CHUNK = 80
EBLK = 2000
TP_PER_CORE = PT // NC
ROWS_PER_TILE = N_PAD // NS
OCHUNK = 64


def _sc_agg_body(xt_hbm, src_hbm, dst_hbm, we_hbm, out_hbm,
                 srcb, dstb, web, gidx0, sidx0, gidx1, sidx1,
                 rows0, rows1, zbuf, accum, semg0, sems0, semg1, sems1):
    c = lax.axis_index("c")
    s = lax.axis_index("s")
    ebase = s * EDGES_PER_TILE
    NCH = EBLK // CHUNK

    z16 = jnp.zeros((16,), jnp.float32)

    def zero_zbuf(r, _):
        for j in range(F // 16):
            zbuf[r, pl.ds(j * 16, 16)] = z16
        return 0

    lax.fori_loop(0, OCHUNK, zero_zbuf, 0)
    for k in range(ROWS_PER_TILE // OCHUNK):
        pltpu.sync_copy(zbuf, accum.at[pl.ds(s * ROWS_PER_TILE + k * OCHUNK,
                                             OCHUNK)])
    plsc.subcore_barrier()

    bufs = ((gidx0, sidx0, rows0, semg0, sems0),
            (gidx1, sidx1, rows1, semg1, sems1))

    def build_idx(ci, gidx, sidx, toff):
        eb = ci * CHUNK
        for j in range(CHUNK // 16):
            sl = pl.ds(j * 16, 16)
            esl = pl.ds(eb + j * 16, 16)
            gidx[sl] = srcb[esl] + toff
            sidx[sl] = dstb[esl]

    def scale_rows(ci, rows):
        eb = ci * CHUNK

        def scale_group(g, _):
            w16 = web[pl.ds(eb + g * 16, 16)]
            for l in range(16):
                w = w16[l]
                r = g * 16 + l
                for j in range(F // 16):
                    sl = pl.ds(j * 16, 16)
                    rows[r, sl] = rows[r, sl] * w
            return 0

        lax.fori_loop(0, CHUNK // 16, scale_group, 0)

    def period_body(tp, _):
        tg = c * TP_PER_CORE + tp
        toff = tg * N_NODES

        def block_body(b, _):
            bb = ebase + b * EBLK
            pltpu.sync_copy(src_hbm.at[pl.ds(bb, EBLK)], srcb)
            pltpu.sync_copy(dst_hbm.at[pl.ds(bb, EBLK)], dstb)
            pltpu.sync_copy(we_hbm.at[pl.ds(bb, EBLK)], web)

            build_idx(0, gidx0, sidx0, toff)
            pltpu.async_copy(xt_hbm.at[gidx0], rows0, semg0)
            pltpu.make_async_copy(xt_hbm.at[gidx0], rows0, semg0).wait()
            build_idx(1, gidx1, sidx1, toff)
            pltpu.async_copy(xt_hbm.at[gidx1], rows1, semg1)
            scale_rows(0, rows0)
            pltpu.async_copy(rows0, accum.at[sidx0], sems0, add=True)

            def pair_body(k2, _):
                for half in range(2):
                    ci = 2 * k2 + 1 + half
                    p = 1 - half
                    gidx, sidx, rows, semg, sems = bufs[p]
                    ngidx, nsidx, nrows, nsemg, nsems = bufs[1 - p]
                    pltpu.make_async_copy(xt_hbm.at[gidx], rows, semg).wait()
                    pltpu.make_async_copy(nrows, accum.at[nsidx],
                                          nsems).wait()

                    @pl.when(ci + 1 < NCH)
                    def _():
                        build_idx(ci + 1, ngidx, nsidx, toff)
                        pltpu.async_copy(xt_hbm.at[ngidx], nrows, nsemg)

                    scale_rows(ci, rows)
                    pltpu.async_copy(rows, accum.at[sidx], sems, add=True)
                return 0

            lax.fori_loop(0, (NCH - 1) // 2, pair_body, 0)
            pltpu.make_async_copy(rows0, accum.at[sidx0], sems0).wait()
            return 0

        lax.fori_loop(0, EDGES_PER_TILE // EBLK, block_body, 0)
        plsc.subcore_barrier()
        for k in range(ROWS_PER_TILE // OCHUNK):
            r0 = s * ROWS_PER_TILE + k * OCHUNK
            pltpu.sync_copy(accum.at[pl.ds(r0, OCHUNK)],
                            out_hbm.at[tg, pl.ds(r0, OCHUNK)])
            pltpu.sync_copy(zbuf, accum.at[pl.ds(r0, OCHUNK)])
        plsc.subcore_barrier()
        return 0

    lax.fori_loop(0, TP_PER_CORE, period_body, 0)


def _sc_aggregate(xt_flat, src, dst, we):
    mesh = plsc.VectorSubcoreMesh(core_axis_name="c", subcore_axis_name="s")
    fn = functools.partial(
        pl.kernel,
        mesh=mesh,
        out_type=jax.ShapeDtypeStruct((PT, N_PAD, F), jnp.float32),
        scratch_types=[
            pltpu.VMEM((EBLK,), jnp.int32),
            pltpu.VMEM((EBLK,), jnp.int32),
            pltpu.VMEM((EBLK,), jnp.float32),
            pltpu.VMEM((CHUNK,), jnp.int32),
            pltpu.VMEM((CHUNK,), jnp.int32),
            pltpu.VMEM((CHUNK,), jnp.int32),
            pltpu.VMEM((CHUNK,), jnp.int32),
            pltpu.VMEM((CHUNK, F), jnp.float32),
            pltpu.VMEM((CHUNK, F), jnp.float32),
            pltpu.VMEM((OCHUNK, F), jnp.float32),
            pltpu.VMEM_SHARED((N_PAD, F), jnp.float32),
            pltpu.SemaphoreType.DMA,
            pltpu.SemaphoreType.DMA,
            pltpu.SemaphoreType.DMA,
            pltpu.SemaphoreType.DMA,
        ],
    )(_sc_agg_body)
    return fn(xt_flat, src, dst, we)


def _gru_block(parts_ref, xt_ref, dinv_ref, probs_ref,
               Wz_ref, Lz_ref, bz_ref, lbz_ref,
               Wr_ref, Lr_ref, br_ref, lbr_ref,
               Wh_ref, Lh_ref, bh_ref, lbh_ref,
               lin_w_ref, lin_b_ref, out_ref):
    C = parts_ref.shape[0]
    f32 = jnp.float32

    def fold(W_ref, L_ref, b_ref, lb_ref):
        Lt = L_ref[0:F, :]
        Lb = L_ref[F:2 * F, :]
        WL = jnp.dot(W_ref[...], Lt, preferred_element_type=f32)
        c = jnp.dot(b_ref[...], Lt, preferred_element_type=f32) + lb_ref[...]
        return WL, Lb, c

    WLz, Lzb, cz = fold(Wz_ref, Lz_ref, bz_ref, lbz_ref)
    WLr, Lrb, cr = fold(Wr_ref, Lr_ref, br_ref, lbr_ref)
    WLh, Lhb, ch = fold(Wh_ref, Lh_ref, bh_ref, lbh_ref)

    dv = dinv_ref[...]
    R = dv.shape[0]
    H = jnp.zeros((R, F), dtype=f32)
    Hacc = jnp.zeros((R, F), dtype=f32)
    for t in range(PT):
        s = parts_ref[0, t]
        for c in range(1, C):
            s = s + parts_ref[c, t]
        a = dv * (s + dv * xt_ref[t])
        Z = jax.nn.sigmoid(jnp.dot(a, WLz, preferred_element_type=f32)
                           + jnp.dot(H, Lzb, preferred_element_type=f32) + cz)
        Rg = jax.nn.sigmoid(jnp.dot(a, WLr, preferred_element_type=f32)
                            + jnp.dot(H, Lrb, preferred_element_type=f32) + cr)
        Ht = jnp.tanh(jnp.dot(a, WLh, preferred_element_type=f32)
                      + jnp.dot(H * Rg, Lhb, preferred_element_type=f32) + ch)
        H = Z * H + (1.0 - Z) * Ht
        Hacc = Hacc + probs_ref[t] * H
    out_ref[...] = (jnp.dot(jnp.maximum(Hacc, 0.0), lin_w_ref[...],
                            preferred_element_type=f32) + lin_b_ref[...])


def _gru_pallas(parts, xt, dinv2d, probs,
                Wz, Lz, bz, lbz, Wr, Lr, br, lbr, Wh, Lh, bh, lbh,
                lin_w, lin_b):
    C = parts.shape[0]
    n = xt.shape[1]
    grid = (pl.cdiv(n, ROW_BLK),)
    full = lambda shape: pl.BlockSpec(shape, lambda i: (0,) * len(shape))
    return pl.pallas_call(
        _gru_block,
        grid=grid,
        in_specs=[
            pl.BlockSpec((C, PT, ROW_BLK, F), lambda i: (0, 0, i, 0)),
            pl.BlockSpec((PT, ROW_BLK, F), lambda i: (0, i, 0)),
            pl.BlockSpec((ROW_BLK, 1), lambda i: (i, 0)),
            pl.BlockSpec(memory_space=pltpu.SMEM),
            full((F, F)), full((2 * F, F)), full((1, F)), full((1, F)),
            full((F, F)), full((2 * F, F)), full((1, F)), full((1, F)),
            full((F, F)), full((2 * F, F)), full((1, F)), full((1, F)),
            full((F, PT)), full((1, PT)),
        ],
        out_specs=pl.BlockSpec((ROW_BLK, PT), lambda i: (i, 0)),
        out_shape=jax.ShapeDtypeStruct((n, PT), jnp.float32),
    )(parts, xt, dinv2d, probs,
      Wz, Lz, bz, lbz, Wr, Lr, br, lbr, Wh, Lh, bh, lbh, lin_w, lin_b)


def kernel(x, edge_index, edge_weight, Wz, bz, Lz, lbz, Wr, br, Lr, lbr,
           Wh, bh, Lh, lbh, att, lin_w, lin_b):
    n = x.shape[0]
    src = edge_index[0]
    dst = edge_index[1]

    deg = jnp.zeros((n,), jnp.float32).at[dst].add(edge_weight) + 1.0
    dinv = jnp.where(deg > 0, lax.rsqrt(jnp.where(deg > 0, deg, 1.0)), 0.0)

    xt_all = jnp.transpose(x, (2, 0, 1))
    we = edge_weight * dinv[src]

    S = _sc_aggregate(xt_all.reshape(PT * n, F), src, dst, we)
    parts = S[None]

    probs = jax.nn.softmax(att)
    out = _gru_pallas(parts, xt_all, dinv.reshape(n, 1), probs,
                      Wz, Lz, bz.reshape(1, F), lbz.reshape(1, F),
                      Wr, Lr, br.reshape(1, F), lbr.reshape(1, F),
                      Wh, Lh, bh.reshape(1, F), lbh.reshape(1, F),
                      lin_w, lin_b.reshape(1, PT))
    return out
